# baseline (speedup 1.0000x reference)
import jax
import jax.numpy as jnp
from jax import lax
from jax.experimental import pallas as pl
from jax.experimental.pallas import tpu as pltpu
from jax.experimental.pallas import tpu_sc as plsc

B = 4096
HIST = 200
HIDDEN = 16
USER_VOCAB = 1000
AD_VOCAB = 1000000
N_USER = 8
N_AD = 5
SEQ_TABLES = (1, 4)

NC = 2
NS = 16

VB = AD_VOCAB // 128
BB = B // 128
SB = HIST // 8
ADP = N_AD * AD_VOCAB * HIDDEN
HBSTRIDE = VB * 8 * 128

TPW = (VB + NS - 1) // NS
SGROUPS = 2 * SB * 8
SPAIRS = SGROUPS // 2


def _iota16():
    return lax.iota(jnp.int32, 16)


def _transpose_R_to_Rt(R, Rt):
    def body(bq, carry):
        bidx = 16 * bq + _iota16()
        for h in range(16):
            col = plsc.load_gather(R, [bidx, jnp.full((16,), h, jnp.int32)])
            Rt[h // 8, h % 8, pl.ds(16 * bq, 16)] = col
        return carry
    lax.fori_loop(0, 8, body, 0)


def _body(ut, adL3, sidx6, uidx3, aidx2,
          u6, a6, s6,
          rm_tab, tb, rb, idxs, R0, R1, Rt0, Rt1, uai, av,
          gsem, wsem, tsem, tw0, gs0, gs1, ws0, ws1):
    cid = lax.axis_index("c")
    sid = lax.axis_index("s")
    wid = cid * NS + sid

    pltpu.sync_copy(uidx3.at[wid], uai)
    for i in range(N_USER):
        Rr = R0 if i % 2 == 0 else R1
        Rtt = Rt0 if i % 2 == 0 else Rt1
        pltpu.async_copy(ut.at[uai.at[i]], Rr, gsem).wait()
        _transpose_R_to_Rt(Rr, Rtt)
        cp0 = pltpu.async_copy(Rtt.at[0], u6.at[i, 0, wid], wsem)
        cp1 = pltpu.async_copy(Rtt.at[1], u6.at[i, 1, wid], wsem)
        cp0.wait()
        cp1.wait()

    pltpu.sync_copy(aidx2.at[wid], av)
    for i in range(N_AD):
        Rtt = Rt0 if i % 2 == 0 else Rt1
        for h in range(16):
            pltpu.async_copy(adL3.at[i, h].at[av.at[i]],
                             Rtt.at[h // 8, h % 8], gsem)
        pltpu.make_async_copy(u6.at[0, 0, 0], Rtt.at[0], gsem).wait()
        pltpu.make_async_copy(u6.at[0, 0, 0], Rtt.at[1], gsem).wait()
        cp0 = pltpu.async_copy(Rtt.at[0], a6.at[i, 0, wid], wsem)
        cp1 = pltpu.async_copy(Rtt.at[1], a6.at[i, 1, wid], wsem)
        cp0.wait()
        cp1.wait()

    t_tab = 1 + cid * 3

    def tbody(k, carry):
        vb = sid + NS * k

        @pl.when(vb < VB)
        def _():
            pltpu.async_copy(
                adL3.at[t_tab, pl.ds(0, 16), pl.ds(128 * vb, 128)],
                tb, tsem).wait()

            @pl.when(k >= 1)
            def _():
                pltpu.make_async_copy(rb, rm_tab.at[cid, pl.ds(0, 128)],
                                      tw0).wait()

            iota = _iota16()
            for v in range(128):
                col = plsc.load_gather(tb, [iota, jnp.full((16,), v,
                                                           jnp.int32)])
                rb[v, :] = col
            pltpu.async_copy(rb, rm_tab.at[cid, pl.ds(128 * vb, 128)], tw0)
        return carry

    lax.fori_loop(0, TPW, tbody, 0)
    pltpu.make_async_copy(rb, rm_tab.at[cid, pl.ds(0, 128)], tw0).wait()

    @pl.when(sid == 0)
    def _():
        pltpu.async_copy(
            adL3.at[t_tab, pl.ds(0, 16), pl.ds(VB * 128, 64)],
            tb.at[pl.ds(0, 16), pl.ds(0, 64)], tsem).wait()
        iota = _iota16()
        for v in range(64):
            col = plsc.load_gather(tb, [iota, jnp.full((16,), v, jnp.int32)])
            rb[v, :] = col
        pltpu.async_copy(rb.at[pl.ds(0, 64)],
                         rm_tab.at[cid, pl.ds(VB * 128, 64)], tw0)
        pltpu.make_async_copy(rb.at[pl.ds(0, 64)],
                              rm_tab.at[cid, pl.ds(0, 64)], tw0).wait()

    plsc.subcore_barrier()

    pltpu.sync_copy(sidx6.at[cid, pl.ds(0, SB), pl.ds(2 * sid, 2)], idxs)

    def coords(g):
        sb = g // 16
        r = g % 16
        return sb, r // 8, r % 8

    def fire(g, Rr, gs):
        sb, b2, si = coords(g)
        pltpu.async_copy(rm_tab.at[cid].at[idxs.at[sb, b2, si]], Rr, gs)

    def process(g, Rr, Rtt, gs, ws):
        sb, b2, si = coords(g)
        pltpu.make_async_copy(rm_tab.at[cid, pl.ds(0, 128)], Rr, gs).wait()
        _transpose_R_to_Rt(Rr, Rtt)

        @pl.when(g >= 2)
        def _():
            pltpu.make_async_copy(Rtt.at[0], u6.at[0, 0, 0], ws).wait()
            pltpu.make_async_copy(Rtt.at[1], u6.at[0, 0, 0], ws).wait()

        sfull = 8 * sb + si
        bb = 2 * sid + b2
        pltpu.async_copy(Rtt.at[0], s6.at[cid, sfull, 0, bb], ws)
        pltpu.async_copy(Rtt.at[1], s6.at[cid, sfull, 1, bb], ws)

    fire(0, R0, gs0)

    def sbody(p, carry):
        g0 = 2 * p
        g1 = 2 * p + 1
        fire(g1, R1, gs1)
        process(g0, R0, Rt0, gs0, ws0)

        @pl.when(g1 + 1 < SGROUPS)
        def _():
            fire(g1 + 1, R0, gs0)
        process(g1, R1, Rt1, gs1, ws1)
        return carry

    lax.fori_loop(0, SPAIRS, sbody, 0)

    pltpu.make_async_copy(Rt0.at[0], u6.at[0, 0, 0], ws0).wait()
    pltpu.make_async_copy(Rt0.at[1], u6.at[0, 0, 0], ws0).wait()
    pltpu.make_async_copy(Rt1.at[0], u6.at[0, 0, 0], ws1).wait()
    pltpu.make_async_copy(Rt1.at[1], u6.at[0, 0, 0], ws1).wait()


def kernel(user_indices, ad_indices, buy_seq_indices, user_tables, ad_tables):
    adL3 = jnp.transpose(ad_tables, (0, 2, 1))
    sidx6 = jnp.transpose(
        buy_seq_indices.reshape(BB, 128, 2, SB, 8), (2, 3, 0, 4, 1)
    )
    uidx3 = jnp.transpose(user_indices.reshape(BB, 128, N_USER), (0, 2, 1))
    uidx3 = uidx3 + (jnp.arange(N_USER, dtype=jnp.int32) * USER_VOCAB
                     )[None, :, None]
    aidx2 = jnp.transpose(ad_indices.reshape(BB, 128, N_AD), (0, 2, 1))
    ut = user_tables.reshape(N_USER * USER_VOCAB, HIDDEN)

    mesh = plsc.VectorSubcoreMesh(core_axis_name="c", subcore_axis_name="s")
    run = pl.kernel(
        _body,
        mesh=mesh,
        out_type=(
            jax.ShapeDtypeStruct((N_USER, 2, BB, 8, 128), jnp.float32),
            jax.ShapeDtypeStruct((N_AD, 2, BB, 8, 128), jnp.float32),
            jax.ShapeDtypeStruct((2, HIST, 2, BB, 8, 128), jnp.float32),
        ),
        scratch_types=[
            pltpu.HBM((2, AD_VOCAB, HIDDEN), jnp.float32),
            pltpu.VMEM((HIDDEN, 128), jnp.float32),
            pltpu.VMEM((128, HIDDEN), jnp.float32),
            pltpu.VMEM((SB, 2, 8, 128), jnp.int32),
            pltpu.VMEM((128, HIDDEN), jnp.float32),
            pltpu.VMEM((128, HIDDEN), jnp.float32),
            pltpu.VMEM((2, 8, 128), jnp.float32),
            pltpu.VMEM((2, 8, 128), jnp.float32),
            pltpu.VMEM((N_USER, 128), jnp.int32),
            pltpu.VMEM((N_AD, 128), jnp.int32),
            pltpu.SemaphoreType.DMA,
            pltpu.SemaphoreType.DMA,
            pltpu.SemaphoreType.DMA,
            pltpu.SemaphoreType.DMA,
            pltpu.SemaphoreType.DMA,
            pltpu.SemaphoreType.DMA,
            pltpu.SemaphoreType.DMA,
            pltpu.SemaphoreType.DMA,
        ],
        compiler_params=pltpu.CompilerParams(use_tc_tiling_on_sc=False,
                                             needs_layout_passes=False),
    )
    u6, a6, s6 = run(ut, adL3, sidx6, uidx3, aidx2)

    user_embs = jnp.transpose(u6, (2, 4, 0, 1, 3)).reshape(B, N_USER, HIDDEN)
    ad_embs = jnp.transpose(a6, (2, 4, 0, 1, 3)).reshape(B, N_AD, HIDDEN)
    buy_seq_embs = jnp.transpose(s6, (3, 5, 0, 1, 2, 4)).reshape(
        B, len(SEQ_TABLES), HIST, HIDDEN)
    return (user_embs, ad_embs, buy_seq_embs)
